# baseline (speedup 1.0000x reference)
import functools

import jax
import jax.numpy as jnp
from jax import lax
from jax.experimental import pallas as pl
from jax.experimental.pallas import tpu as pltpu
from jax.experimental.pallas import tpu_sc as plsc

N = 10000
E = 320000
D = 128
H = 16
C = 2
G = 64

NC = 2
NS = 16
NW = NC * NS
EPW = E // NW
CHUNK = 125
NCHUNK = EPW // CHUNK
NPAD = 10240
ROWS_PT = NPAD // NS

_MESH = plsc.VectorSubcoreMesh(core_axis_name="c", subcore_axis_name="s")
_SC_PARAMS = pltpu.CompilerParams(
    use_tc_tiling_on_sc=False, needs_layout_passes=False
)


def _rsqrt16(d):
    bits = plsc.bitcast(d, jnp.int32)
    y = plsc.bitcast(jnp.int32(0x5F3759DF) - (bits >> 1), jnp.float32)
    for _ in range(3):
        y = y * (1.5 - 0.5 * d * y * y)
    return y


@functools.partial(
    pl.kernel,
    out_type=(
        jax.ShapeDtypeStruct((NC, NPAD, H), jnp.float32),
        jax.ShapeDtypeStruct((NPAD,), jnp.float32),
    ),
    mesh=_MESH,
    compiler_params=_SC_PARAMS,
    scratch_types=[
        pltpu.VMEM((NCHUNK, CHUNK), jnp.int32),
        pltpu.VMEM((NCHUNK, CHUNK), jnp.int32),
        pltpu.VMEM((ROWS_PT, H), jnp.float32),
        pltpu.VMEM((ROWS_PT,), jnp.float32),
        pltpu.VMEM((CHUNK,), jnp.float32),
        pltpu.VMEM((CHUNK, H), jnp.float32),
        pltpu.VMEM((CHUNK, H), jnp.float32),
        pltpu.VMEM_SHARED((NPAD,), jnp.float32),
        pltpu.VMEM_SHARED((NPAD, H), jnp.float32),
        pltpu.VMEM_SHARED((NPAD, H), jnp.float32),
        pltpu.SemaphoreType.DMA,
        pltpu.SemaphoreType.DMA,
    ],
)
def _gcn_sc(src_hbm, dst_hbm, h_hbm, ones_hbm, zrow_hbm, zrows_hbm,
            t_hbm, dinv_hbm,
            src_v, dst_v, hrows_v, dinv_v, ones_v, rows0, rows1,
            shared_deg, shared_hp, shared_t, sem0, sem1):
    cid = lax.axis_index("c")
    sid = lax.axis_index("s")
    wid = sid * NC + cid
    myrows = pl.ds(sid * ROWS_PT, ROWS_PT)

    pltpu.sync_copy(dst_hbm.at[wid], dst_v)
    pltpu.sync_copy(ones_hbm, ones_v)
    pltpu.sync_copy(h_hbm.at[myrows], hrows_v)
    pltpu.sync_copy(zrow_hbm, shared_deg.at[myrows])
    plsc.subcore_barrier()

    def deg_body(j, carry):
        pltpu.sync_copy(ones_v, shared_deg.at[dst_v.at[j]], add=True)
        return carry

    lax.fori_loop(0, NCHUNK, deg_body, 0)
    pltpu.sync_copy(dst_hbm.at[sid * NC + (1 - cid)], src_v)

    def deg_body2(j, carry):
        pltpu.sync_copy(ones_v, shared_deg.at[src_v.at[j]], add=True)
        return carry

    lax.fori_loop(0, NCHUNK, deg_body2, 0)
    plsc.subcore_barrier()
    pltpu.sync_copy(src_hbm.at[wid], src_v)

    pltpu.sync_copy(shared_deg.at[myrows], dinv_v)

    def dinv_body(k, carry):
        d = dinv_v[pl.ds(k * 16, 16)] + 1.0
        dinv_v[pl.ds(k * 16, 16)] = _rsqrt16(d)
        return carry

    lax.fori_loop(0, ROWS_PT // 16, dinv_body, 0)

    def scale_body(k, carry):
        dv = dinv_v[pl.ds(k * 16, 16)]
        for i in range(16):
            r = k * 16 + i
            hrows_v[r, :] = hrows_v[r, :] * dv[i]
        return carry

    lax.fori_loop(0, ROWS_PT // 16, scale_body, 0)

    pltpu.sync_copy(hrows_v, shared_hp.at[myrows])

    @pl.when(cid == 0)
    def _():
        pltpu.sync_copy(hrows_v, shared_t.at[myrows])
        pltpu.sync_copy(dinv_v, dinv_hbm.at[myrows])

    @pl.when(cid != 0)
    def _():
        pltpu.sync_copy(zrows_hbm, shared_t.at[myrows])

    plsc.subcore_barrier()

    pltpu.async_copy(shared_hp.at[src_v.at[0]], rows0, sem0)

    def edge_body(i, carry):
        j = 2 * i
        pltpu.async_copy(shared_hp.at[src_v.at[j + 1]], rows1, sem1)
        pltpu.make_async_copy(shared_hp.at[src_v.at[j]], rows0, sem0).wait()
        pltpu.sync_copy(rows0, shared_t.at[dst_v.at[j]], add=True)

        @pl.when(j + 2 < NCHUNK)
        def _():
            pltpu.async_copy(shared_hp.at[src_v.at[j + 2]], rows0, sem0)

        pltpu.make_async_copy(shared_hp.at[src_v.at[j + 1]], rows1, sem1).wait()
        pltpu.sync_copy(rows1, shared_t.at[dst_v.at[j + 1]], add=True)
        return carry

    lax.fori_loop(0, NCHUNK // 2, edge_body, 0)
    plsc.subcore_barrier()
    pltpu.sync_copy(shared_t.at[myrows], t_hbm.at[cid, myrows])


def _mm_tc(x_ref, w_ref, h_ref):
    h = jnp.dot(x_ref[...], w_ref[...], preferred_element_type=jnp.float32)
    h_ref[0:N, :] = h
    h_ref[N:NPAD, :] = jnp.zeros((NPAD - N, H), jnp.float32)


_mm_call = pl.pallas_call(
    _mm_tc,
    out_shape=jax.ShapeDtypeStruct((NPAD, H), jnp.float32),
)


def _final_tc(t_ref, dinv_ref, b_ref, batch_ref, wc_ref, bc_ref, out_ref):
    agg = dinv_ref[...] * (t_ref[0] + t_ref[1]) + b_ref[...]
    r = jnp.maximum(agg, 0.0)
    gids = lax.broadcasted_iota(jnp.int32, (G, NPAD), 0)
    onehot = (batch_ref[...] == gids).astype(jnp.float32)
    sums = jnp.dot(onehot, r, preferred_element_type=jnp.float32)
    counts = jnp.sum(onehot, axis=1, keepdims=True)
    pooled = sums / jnp.maximum(counts, 1.0)
    out_ref[...] = (
        jnp.dot(pooled, wc_ref[...], preferred_element_type=jnp.float32)
        + bc_ref[...]
    )


_final_call = pl.pallas_call(
    _final_tc,
    out_shape=jax.ShapeDtypeStruct((G, C), jnp.float32),
)


def kernel(x, edge_index, batch, W, b, Wc, bc):
    src3 = edge_index[0].reshape(NW, NCHUNK, CHUNK)
    dst3 = edge_index[1].reshape(NW, NCHUNK, CHUNK)
    ones = jnp.ones((CHUNK,), jnp.float32)
    zrow = jnp.zeros((ROWS_PT,), jnp.float32)
    zrows = jnp.zeros((ROWS_PT, H), jnp.float32)

    h = _mm_call(x, W)
    t, dinv = _gcn_sc(src3, dst3, h, ones, zrow, zrows)
    dinv = dinv.reshape(NPAD, 1)

    batpad = jnp.pad(batch, (0, NPAD - N), constant_values=G).reshape(1, NPAD)
    logits = _final_call(t, dinv, b.reshape(1, H), batpad, Wc,
                         bc.reshape(1, C))
    return logits


# speedup vs baseline: 1.0822x; 1.0822x over previous
import functools

import jax
import jax.numpy as jnp
from jax import lax
from jax.experimental import pallas as pl
from jax.experimental.pallas import tpu as pltpu
from jax.experimental.pallas import tpu_sc as plsc

N = 10000
E = 320000
D = 128
H = 16
C = 2
G = 64

NC = 2
NS = 16
NW = NC * NS
EPW = E // NW
CHUNK = 125
NCHUNK = EPW // CHUNK
NPAD = 10240
ROWS_PT = NPAD // NS

_MESH = plsc.VectorSubcoreMesh(core_axis_name="c", subcore_axis_name="s")
_SC_PARAMS = pltpu.CompilerParams(
    use_tc_tiling_on_sc=False, needs_layout_passes=False
)


def _rsqrt16(d):
    bits = plsc.bitcast(d, jnp.int32)
    y = plsc.bitcast(jnp.int32(0x5F3759DF) - (bits >> 1), jnp.float32)
    for _ in range(3):
        y = y * (1.5 - 0.5 * d * y * y)
    return y


@functools.partial(
    pl.kernel,
    out_type=(
        jax.ShapeDtypeStruct((NC, NPAD, H), jnp.float32),
        jax.ShapeDtypeStruct((NPAD,), jnp.float32),
    ),
    mesh=_MESH,
    compiler_params=_SC_PARAMS,
    scratch_types=[
        pltpu.VMEM((NCHUNK, CHUNK), jnp.int32),
        pltpu.VMEM((NCHUNK, CHUNK), jnp.int32),
        pltpu.VMEM((NCHUNK, CHUNK), jnp.int32),
        pltpu.VMEM((ROWS_PT, H), jnp.float32),
        pltpu.VMEM((ROWS_PT,), jnp.float32),
        pltpu.VMEM((CHUNK,), jnp.float32),
        [pltpu.VMEM((CHUNK, H), jnp.float32)] * 4,
        pltpu.SemaphoreType.DMA,
        pltpu.SemaphoreType.DMA,
        [pltpu.SemaphoreType.DMA] * 4,
        [pltpu.SemaphoreType.DMA] * 4,
        pltpu.VMEM_SHARED((NPAD,), jnp.float32),
        pltpu.VMEM_SHARED((NPAD, H), jnp.float32),
        pltpu.VMEM_SHARED((NPAD, H), jnp.float32),
    ],
)
def _gcn_sc(src_hbm, dst_hbm, h_hbm, ones_hbm, zrow_hbm, zrows_hbm,
            t_hbm, dinv_hbm,
            src_v, dst_v, dstb_v, hrows_v, dinv_v, ones_v, rows,
            semd, semld, gsems, ssems,
            shared_deg, shared_hp, shared_t):
    cid = lax.axis_index("c")
    sid = lax.axis_index("s")
    wid = sid * NC + cid
    myrows = pl.ds(sid * ROWS_PT, ROWS_PT)

    lda = pltpu.async_copy(dst_hbm.at[wid], dst_v, gsems[0])
    ldb = pltpu.async_copy(dst_hbm.at[sid * NC + (1 - cid)], dstb_v, gsems[1])
    lds = pltpu.async_copy(src_hbm.at[wid], src_v, gsems[2])
    ldh = pltpu.async_copy(h_hbm.at[myrows], hrows_v, gsems[3])
    pltpu.sync_copy(ones_hbm, ones_v)
    pltpu.sync_copy(zrow_hbm, shared_deg.at[myrows])
    lda.wait()
    ldb.wait()
    plsc.subcore_barrier()

    DEG_DEPTH = 4

    def deg_pass(idxref):
        for j in range(DEG_DEPTH):
            pltpu.async_copy(ones_v, shared_deg.at[idxref.at[j]], semd,
                             add=True)

        def body(j, carry):
            pltpu.make_async_copy(ones_v, shared_deg.at[idxref.at[0]],
                                  semd).wait()
            pltpu.async_copy(ones_v, shared_deg.at[idxref.at[j]], semd,
                             add=True)
            return carry

        lax.fori_loop(DEG_DEPTH, NCHUNK, body, 0)
        for _ in range(DEG_DEPTH):
            pltpu.make_async_copy(ones_v, shared_deg.at[idxref.at[0]],
                                  semd).wait()

    deg_pass(dst_v)
    deg_pass(dstb_v)
    lds.wait()
    ldh.wait()
    plsc.subcore_barrier()

    pltpu.sync_copy(shared_deg.at[myrows], dinv_v)

    def dinv_body(k, carry):
        d = dinv_v[pl.ds(k * 16, 16)] + 1.0
        dinv_v[pl.ds(k * 16, 16)] = _rsqrt16(d)
        return carry

    lax.fori_loop(0, ROWS_PT // 16, dinv_body, 0)

    def scale_body(k, carry):
        dv = dinv_v[pl.ds(k * 16, 16)]
        for i in range(16):
            r = k * 16 + i
            hrows_v[r, :] = hrows_v[r, :] * dv[i]
        return carry

    lax.fori_loop(0, ROWS_PT // 16, scale_body, 0)

    pltpu.sync_copy(hrows_v, shared_hp.at[myrows])

    @pl.when(cid == 0)
    def _():
        pltpu.sync_copy(hrows_v, shared_t.at[myrows])
        pltpu.sync_copy(dinv_v, dinv_hbm.at[myrows])

    @pl.when(cid != 0)
    def _():
        pltpu.sync_copy(zrows_hbm, shared_t.at[myrows])

    plsc.subcore_barrier()

    for b in range(4):
        pltpu.async_copy(shared_hp.at[src_v.at[b]], rows[b], gsems[b])

    def edge_body(i, carry):
        j = 4 * i
        for b in range(4):
            pltpu.make_async_copy(shared_hp.at[src_v.at[j + b]], rows[b],
                                  gsems[b]).wait()
            pltpu.async_copy(rows[b], shared_t.at[dst_v.at[j + b]], ssems[b],
                             add=True)
        for b in range(4):
            pltpu.make_async_copy(rows[b], shared_t.at[dst_v.at[j + b]],
                                  ssems[b]).wait()

            @pl.when(j + 4 + b < NCHUNK)
            def _():
                pltpu.async_copy(shared_hp.at[src_v.at[j + 4 + b]], rows[b],
                                 gsems[b])
        return carry

    lax.fori_loop(0, NCHUNK // 4, edge_body, 0)
    plsc.subcore_barrier()
    pltpu.sync_copy(shared_t.at[myrows], t_hbm.at[cid, myrows])


def _mm_tc(x_ref, w_ref, h_ref):
    h = jnp.dot(x_ref[...], w_ref[...], preferred_element_type=jnp.float32)
    h_ref[0:N, :] = h
    h_ref[N:NPAD, :] = jnp.zeros((NPAD - N, H), jnp.float32)


_mm_call = pl.pallas_call(
    _mm_tc,
    out_shape=jax.ShapeDtypeStruct((NPAD, H), jnp.float32),
)


def _final_tc(t_ref, dinv_ref, b_ref, batch_ref, wc_ref, bc_ref, out_ref):
    agg = dinv_ref[...] * (t_ref[0] + t_ref[1]) + b_ref[...]
    r = jnp.maximum(agg, 0.0)
    gids = lax.broadcasted_iota(jnp.int32, (G, NPAD), 0)
    onehot = (batch_ref[...] == gids).astype(jnp.float32)
    sums = jnp.dot(onehot, r, preferred_element_type=jnp.float32)
    counts = jnp.sum(onehot, axis=1, keepdims=True)
    pooled = sums / jnp.maximum(counts, 1.0)
    out_ref[...] = (
        jnp.dot(pooled, wc_ref[...], preferred_element_type=jnp.float32)
        + bc_ref[...]
    )


_final_call = pl.pallas_call(
    _final_tc,
    out_shape=jax.ShapeDtypeStruct((G, C), jnp.float32),
)


def kernel(x, edge_index, batch, W, b, Wc, bc):
    src3 = edge_index[0].reshape(NW, NCHUNK, CHUNK)
    dst3 = edge_index[1].reshape(NW, NCHUNK, CHUNK)
    ones = jnp.ones((CHUNK,), jnp.float32)
    zrow = jnp.zeros((ROWS_PT,), jnp.float32)
    zrows = jnp.zeros((ROWS_PT, H), jnp.float32)

    h = _mm_call(x, W)
    t, dinv = _gcn_sc(src3, dst3, h, ones, zrow, zrows)
    dinv = dinv.reshape(NPAD, 1)

    batpad = jnp.pad(batch, (0, NPAD - N), constant_values=G).reshape(1, NPAD)
    logits = _final_call(t, dinv, b.reshape(1, H), batpad, Wc,
                         bc.reshape(1, C))
    return logits


# speedup vs baseline: 1.2872x; 1.1895x over previous
import functools

import jax
import jax.numpy as jnp
from jax import lax
from jax.experimental import pallas as pl
from jax.experimental.pallas import tpu as pltpu
from jax.experimental.pallas import tpu_sc as plsc

N = 10000
E = 320000
D = 128
H = 16
C = 2
G = 64

NC = 2
NS = 16
NW = NC * NS
EPW = E // NW
CHUNK = 125
NCHUNK = EPW // CHUNK
NPAD = 10240
ROWS_PT = NPAD // NS

_MESH = plsc.VectorSubcoreMesh(core_axis_name="c", subcore_axis_name="s")
_SC_PARAMS = pltpu.CompilerParams(
    use_tc_tiling_on_sc=False, needs_layout_passes=False
)


def _rsqrt16(d):
    bits = plsc.bitcast(d, jnp.int32)
    y = plsc.bitcast(jnp.int32(0x5F3759DF) - (bits >> 1), jnp.float32)
    for _ in range(3):
        y = y * (1.5 - 0.5 * d * y * y)
    return y


@functools.partial(
    pl.kernel,
    out_type=jax.ShapeDtypeStruct((NC, NPAD, H), jnp.float32),
    mesh=_MESH,
    compiler_params=_SC_PARAMS,
    scratch_types=[
        pltpu.VMEM((NCHUNK, CHUNK), jnp.int32),
        pltpu.VMEM((NCHUNK, CHUNK), jnp.int32),
        pltpu.VMEM((NCHUNK, CHUNK), jnp.int32),
        pltpu.VMEM((ROWS_PT, H), jnp.float32),
        pltpu.VMEM((ROWS_PT,), jnp.float32),
        pltpu.VMEM((CHUNK,), jnp.float32),
        [pltpu.VMEM((CHUNK, H), jnp.float32)] * 4,
        pltpu.SemaphoreType.DMA,
        [pltpu.SemaphoreType.DMA] * 4,
        [pltpu.SemaphoreType.DMA] * 4,
        pltpu.VMEM_SHARED((NPAD,), jnp.float32),
        pltpu.VMEM_SHARED((NPAD, H), jnp.float32),
        pltpu.VMEM_SHARED((NPAD, H), jnp.float32),
    ],
)
def _gcn_sc(edge_hbm, h_hbm, ones_hbm, zrow_hbm, zrows_hbm, t_hbm,
            src_v, dst_v, dstb_v, hrows_v, dinv_v, ones_v, rows,
            semd, gsems, ssems,
            shared_deg, shared_hp, shared_t):
    cid = lax.axis_index("c")
    sid = lax.axis_index("s")
    wid = sid * NC + cid
    myrows = pl.ds(sid * ROWS_PT, ROWS_PT)

    lda = pltpu.async_copy(edge_hbm.at[1, wid], dst_v, gsems[0])
    ldb = pltpu.async_copy(edge_hbm.at[1, sid * NC + (1 - cid)], dstb_v,
                           gsems[1])
    lds = pltpu.async_copy(edge_hbm.at[0, wid], src_v, gsems[2])
    ldh = pltpu.async_copy(h_hbm.at[myrows], hrows_v, gsems[3])
    pltpu.sync_copy(ones_hbm, ones_v)
    pltpu.sync_copy(zrow_hbm, shared_deg.at[myrows])
    lda.wait()
    ldb.wait()
    plsc.subcore_barrier()

    DEG_DEPTH = 4

    def deg_pass(idxref):
        for j in range(DEG_DEPTH):
            pltpu.async_copy(ones_v, shared_deg.at[idxref.at[j]], semd,
                             add=True)

        def body(j, carry):
            pltpu.make_async_copy(ones_v, shared_deg.at[idxref.at[0]],
                                  semd).wait()
            pltpu.async_copy(ones_v, shared_deg.at[idxref.at[j]], semd,
                             add=True)
            return carry

        lax.fori_loop(DEG_DEPTH, NCHUNK, body, 0)
        for _ in range(DEG_DEPTH):
            pltpu.make_async_copy(ones_v, shared_deg.at[idxref.at[0]],
                                  semd).wait()

    deg_pass(dst_v)
    deg_pass(dstb_v)
    lds.wait()
    ldh.wait()
    plsc.subcore_barrier()

    pltpu.sync_copy(shared_deg.at[myrows], dinv_v)

    def dinv_body(k, carry):
        d = dinv_v[pl.ds(k * 16, 16)] + 1.0
        dinv_v[pl.ds(k * 16, 16)] = _rsqrt16(d)
        return carry

    lax.fori_loop(0, ROWS_PT // 16, dinv_body, 0)

    def scale_body(k, carry):
        dv = dinv_v[pl.ds(k * 16, 16)]
        for i in range(16):
            r = k * 16 + i
            hrows_v[r, :] = hrows_v[r, :] * dv[i]
        return carry

    lax.fori_loop(0, ROWS_PT // 16, scale_body, 0)

    pltpu.sync_copy(hrows_v, shared_hp.at[myrows])

    @pl.when(cid == 0)
    def _():
        pltpu.sync_copy(hrows_v, shared_t.at[myrows])

    @pl.when(cid != 0)
    def _():
        pltpu.sync_copy(zrows_hbm, shared_t.at[myrows])

    plsc.subcore_barrier()

    for b in range(4):
        pltpu.async_copy(shared_hp.at[src_v.at[b]], rows[b], gsems[b])

    def edge_body(i, carry):
        j = 4 * i
        for b in range(4):
            pltpu.make_async_copy(shared_hp.at[src_v.at[j + b]], rows[b],
                                  gsems[b]).wait()
            pltpu.async_copy(rows[b], shared_t.at[dst_v.at[j + b]], ssems[b],
                             add=True)
        for b in range(4):
            pltpu.make_async_copy(rows[b], shared_t.at[dst_v.at[j + b]],
                                  ssems[b]).wait()

            @pl.when(j + 4 + b < NCHUNK)
            def _():
                pltpu.async_copy(shared_hp.at[src_v.at[j + 4 + b]], rows[b],
                                 gsems[b])
        return carry

    lax.fori_loop(0, NCHUNK // 4, edge_body, 0)
    plsc.subcore_barrier()

    pltpu.sync_copy(shared_t.at[myrows], hrows_v)

    def scale_t_body(k, carry):
        dv = dinv_v[pl.ds(k * 16, 16)]
        for i in range(16):
            r = k * 16 + i
            hrows_v[r, :] = hrows_v[r, :] * dv[i]
        return carry

    lax.fori_loop(0, ROWS_PT // 16, scale_t_body, 0)
    pltpu.sync_copy(hrows_v, t_hbm.at[cid, myrows])


def _mm_tc(x_ref, w_ref, h_ref):
    h = jnp.dot(x_ref[...], w_ref[...], preferred_element_type=jnp.float32)
    h_ref[0:N, :] = h
    h_ref[N:NPAD, :] = jnp.zeros((NPAD - N, H), jnp.float32)


_mm_call = pl.pallas_call(
    _mm_tc,
    out_shape=jax.ShapeDtypeStruct((NPAD, H), jnp.float32),
)


def _final_tc(t_ref, b_ref, batch_ref, wc_ref, bc_ref, out_ref):
    agg = t_ref[0, 0:N, :] + t_ref[1, 0:N, :] + b_ref[...]
    r = jnp.maximum(agg, 0.0)
    gids = lax.broadcasted_iota(jnp.int32, (G, N), 0)
    onehot = (batch_ref[...] == gids).astype(jnp.float32)
    sums = jnp.dot(onehot, r, preferred_element_type=jnp.float32)
    counts = jnp.sum(onehot, axis=1, keepdims=True)
    pooled = sums / jnp.maximum(counts, 1.0)
    out_ref[...] = (
        jnp.dot(pooled, wc_ref[...], preferred_element_type=jnp.float32)
        + bc_ref[...]
    )


_final_call = pl.pallas_call(
    _final_tc,
    out_shape=jax.ShapeDtypeStruct((G, C), jnp.float32),
)


def kernel(x, edge_index, batch, W, b, Wc, bc):
    e4 = edge_index.reshape(2, NW, NCHUNK, CHUNK)
    ones = jnp.ones((CHUNK,), jnp.float32)
    zrow = jnp.zeros((ROWS_PT,), jnp.float32)
    zrows = jnp.zeros((ROWS_PT, H), jnp.float32)

    h = _mm_call(x, W)
    t = _gcn_sc(e4, h, ones, zrow, zrows)

    logits = _final_call(t, b.reshape(1, H), batch.reshape(1, N), Wc,
                         bc.reshape(1, C))
    return logits
